# Initial kernel scaffold; baseline (speedup 1.0000x reference)
import jax
import jax.numpy as jnp
from jax.experimental import pallas as pl


def kernel(x, edge_index, W1_l, b1_l, W1_r, gamma, beta, W2_l, b2_l, W2_r):
    raise NotImplementedError("write your pallas kernel here")


# speedup vs baseline: 4.6909x; 4.6909x over previous
import functools

import jax
import jax.numpy as jnp
from jax import lax
from jax.experimental import pallas as pl
from jax.experimental.pallas import tpu as pltpu
from jax.experimental.pallas import tpu_sc as plsc

N = 10000
H = 128
LANES = 128
NW = 32
N_PAD = 10240
RPT = N_PAD // 16
DEAD = N_PAD - 8
BLK = 400


def _k1_body(x_ref, w_ref, b_ref, yg_ref, z_ref):
    xw = jnp.dot(x_ref[...], w_ref[...], preferred_element_type=jnp.float32)
    yg_ref[...] = xw[:, :H]
    z_ref[...] = xw[:, H:] + b_ref[...]


def _k2_body(agg_ref, dp_ref, z1_ref, g_ref, be_ref, w_ref, b2_ref,
             yg2_ref, z2_ref, rdeg_ref):
    p = agg_ref[...]
    dp = dp_ref[...]
    ssum = p[0] + p[1]
    deg = dp[0, :, 0:1] + dp[1, :, 0:1]
    rdeg = 1.0 / jnp.maximum(deg, 1.0)
    rdeg_ref[...] = jnp.broadcast_to(rdeg, (rdeg.shape[0], 16))
    pre = ssum * rdeg + z1_ref[...]
    mu = jnp.mean(pre, axis=-1, keepdims=True)
    d = pre - mu
    var = jnp.mean(d * d, axis=-1, keepdims=True)
    h = d * lax.rsqrt(var + 1e-5) * g_ref[...] + be_ref[...]
    h = jnp.maximum(h, 0.0)
    hw = jnp.dot(h, w_ref[...], preferred_element_type=jnp.float32)
    yg2_ref[...] = hw[:, :H]
    z2_ref[...] = hw[:, H:] + b2_ref[...]


def _k3_body(agg2_ref, rdeg_ref, z2_ref, out_ref):
    p = agg2_ref[...]
    rdeg = rdeg_ref[...][:, 0:1]
    out_ref[...] = (p[0] + p[1]) * rdeg + z2_ref[...]


def _sc_mesh():
    return plsc.VectorSubcoreMesh(
        core_axis_name="c", subcore_axis_name="s", num_cores=2,
        num_subcores=16)


def _make_sc_agg(n_chunks):
    @functools.partial(
        pl.kernel,
        out_type=jax.ShapeDtypeStruct((2, N_PAD, H), jnp.float32),
        mesh=_sc_mesh(),
        scratch_types=[
            pltpu.VMEM((n_chunks, LANES), jnp.int32),
            pltpu.VMEM((n_chunks, LANES), jnp.int32),
            pltpu.VMEM((LANES, H), jnp.float32),
            pltpu.VMEM_SHARED((N_PAD, H), jnp.float32),
            pltpu.SemaphoreType.DMA,
        ],
    )
    def sc_agg(y_hbm, src_hbm, dst_hbm, zeros_hbm, out_hbm,
               src_v, dst_v, rows_v, acc_sh, sem):
        c = lax.axis_index("c")
        s = lax.axis_index("s")
        wid = s * 2 + c
        pltpu.sync_copy(zeros_hbm, acc_sh.at[pl.ds(s * RPT, RPT)])
        pltpu.sync_copy(src_hbm.at[wid], src_v)
        pltpu.sync_copy(dst_hbm.at[wid], dst_v)
        plsc.subcore_barrier()

        def chunk(j, carry):
            pltpu.async_copy(y_hbm.at[src_v.at[j]], rows_v, sem).wait()
            pltpu.sync_copy(rows_v, acc_sh.at[dst_v.at[j]], add=True)
            return carry

        lax.fori_loop(0, n_chunks, chunk, 0)
        plsc.subcore_barrier()
        pltpu.sync_copy(acc_sh.at[pl.ds(s * RPT, RPT)],
                        out_hbm.at[c, pl.ds(s * RPT, RPT)])

    return sc_agg


def _make_sc_deg(n_chunks):
    @functools.partial(
        pl.kernel,
        out_type=jax.ShapeDtypeStruct((2, N_PAD, H), jnp.float32),
        mesh=_sc_mesh(),
        scratch_types=[
            pltpu.VMEM((n_chunks, LANES), jnp.int32),
            pltpu.VMEM((LANES, H), jnp.float32),
            pltpu.VMEM_SHARED((N_PAD, H), jnp.float32),
        ],
    )
    def sc_deg(dst_hbm, ones_hbm, zeros_hbm, out_hbm, dst_v, ones_v, acc_sh):
        c = lax.axis_index("c")
        s = lax.axis_index("s")
        wid = s * 2 + c
        pltpu.sync_copy(zeros_hbm, acc_sh.at[pl.ds(s * RPT, RPT)])
        pltpu.sync_copy(ones_hbm, ones_v)
        pltpu.sync_copy(dst_hbm.at[wid], dst_v)
        plsc.subcore_barrier()

        def chunk(j, carry):
            pltpu.sync_copy(ones_v, acc_sh.at[dst_v.at[j]], add=True)
            return carry

        lax.fori_loop(0, n_chunks, chunk, 0)
        plsc.subcore_barrier()
        pltpu.sync_copy(acc_sh.at[pl.ds(s * RPT, RPT)],
                        out_hbm.at[c, pl.ds(s * RPT, RPT)])

    return sc_deg


def _tc_call(body, in_arrays, in_specs, out_shapes, out_specs, grid):
    return pl.pallas_call(
        body, grid=grid, in_specs=in_specs,
        out_specs=out_specs, out_shape=out_shapes,
    )(*in_arrays)


def kernel(x, edge_index, W1_l, b1_l, W1_r, gamma, beta, W2_l, b2_l, W2_r):
    n, in_dim = x.shape
    e = edge_index.shape[1]
    n_chunks = -(-e // (NW * LANES))
    e_pad = NW * n_chunks * LANES

    ei = edge_index.astype(jnp.int32)
    pad = e_pad - e
    src3d = jnp.concatenate(
        [ei[0], jnp.zeros((pad,), jnp.int32)]).reshape(NW, n_chunks, LANES)
    dst3d = jnp.concatenate(
        [ei[1], jnp.full((pad,), DEAD, jnp.int32)]).reshape(NW, n_chunks, LANES)
    zeros_rp = jnp.zeros((RPT, H), jnp.float32)
    ones_blk = jnp.ones((LANES, H), jnp.float32)
    wt1 = jnp.concatenate([W1_l, W1_r], axis=0).T
    wt2 = jnp.concatenate([W2_l, W2_r], axis=0).T
    b1r = b1_l.reshape(1, H)
    b2r = b2_l.reshape(1, H)
    gr = gamma.reshape(1, H)
    br = beta.reshape(1, H)

    grid = (n // BLK,)
    row_spec = lambda w: pl.BlockSpec((BLK, w), lambda i: (i, 0))
    full_spec = lambda a: pl.BlockSpec(a.shape, lambda i: (0, 0))
    part_spec = pl.BlockSpec((2, BLK, H), lambda i: (0, i, 0))

    yg1, z1 = _tc_call(
        _k1_body, (x, wt1, b1r),
        [row_spec(in_dim), full_spec(wt1), full_spec(b1r)],
        [jax.ShapeDtypeStruct((n, H), jnp.float32),
         jax.ShapeDtypeStruct((n, H), jnp.float32)],
        [row_spec(H), row_spec(H)], grid)

    degp = _make_sc_deg(n_chunks)(dst3d, ones_blk, zeros_rp)
    agg1 = _make_sc_agg(n_chunks)(yg1, src3d, dst3d, zeros_rp)

    yg2, z2, rdeg = _tc_call(
        _k2_body, (agg1, degp, z1, gr, br, wt2, b2r),
        [part_spec, part_spec, row_spec(H), full_spec(gr), full_spec(br),
         full_spec(wt2), full_spec(b2r)],
        [jax.ShapeDtypeStruct((n, H), jnp.float32),
         jax.ShapeDtypeStruct((n, H), jnp.float32),
         jax.ShapeDtypeStruct((n, 16), jnp.float32)],
        [row_spec(H), row_spec(H), row_spec(16)], grid)

    agg2 = _make_sc_agg(n_chunks)(yg2, src3d, dst3d, zeros_rp)

    out = _tc_call(
        _k3_body, (agg2, rdeg, z2),
        [part_spec, row_spec(16), row_spec(H)],
        jax.ShapeDtypeStruct((n, H), jnp.float32),
        row_spec(H), grid)
    return out
